# baseline (speedup 1.0000x reference)
import functools

import jax
import jax.numpy as jnp
from jax import lax
from jax.experimental import pallas as pl
from jax.experimental.pallas import tpu as pltpu
from jax.experimental.pallas import tpu_sc as plsc

NUM_CLASSES = 1000
DIM = 128
BATCH = 16384

_info = plsc.get_sparse_core_info()
_NC, _NS = _info.num_cores, _info.num_subcores
_NW = _NC * _NS
_B_PER_W = BATCH // _NW
_CHUNK = 128
_NCHUNK = _B_PER_W // _CHUNK
_VPAD = 1024
_ROWS_PER_TILE = _VPAD // _NS


def _gather_body(y_hbm, table_hbm, out_hbm, idx_v, rows_v, stage_v, table_sh,
                 *sems):
    gsems = sems[:_NCHUNK]
    wsem = sems[_NCHUNK]
    cid = lax.axis_index("c")
    sid = lax.axis_index("s")
    wid = sid * _NC + cid
    base = wid * _B_PER_W
    shard = pl.ds(sid * _ROWS_PER_TILE, _ROWS_PER_TILE)
    pltpu.sync_copy(table_hbm.at[shard], stage_v)
    pltpu.sync_copy(stage_v, table_sh.at[shard])
    pltpu.sync_copy(y_hbm.at[pl.ds(wid * _NCHUNK, _NCHUNK)], idx_v)
    plsc.subcore_barrier()
    gathers = [
        pltpu.async_copy(table_sh.at[idx_v.at[j]], rows_v.at[j], gsems[j])
        for j in range(_NCHUNK)
    ]
    writes = []
    for j in range(_NCHUNK):
        gathers[j].wait()
        writes.append(
            pltpu.async_copy(
                rows_v.at[j], out_hbm.at[pl.ds(base + j * _CHUNK, _CHUNK)], wsem
            )
        )
    for w in writes:
        w.wait()


def kernel(y, table):
    mesh = plsc.VectorSubcoreMesh(core_axis_name="c", subcore_axis_name="s")
    k = functools.partial(
        pl.kernel,
        mesh=mesh,
        out_type=jax.ShapeDtypeStruct((BATCH, DIM), jnp.float32),
        scratch_types=[
            pltpu.VMEM((_NCHUNK, _CHUNK), jnp.int32),
            pltpu.VMEM((_NCHUNK, _CHUNK, DIM), jnp.float32),
            pltpu.VMEM((_ROWS_PER_TILE, DIM), jnp.float32),
            pltpu.VMEM_SHARED((_VPAD, DIM), jnp.float32),
        ]
        + [pltpu.SemaphoreType.DMA] * (_NCHUNK + 1),
    )(_gather_body)
    y2d = y.astype(jnp.int32).reshape(_NW * _NCHUNK, _CHUNK)
    table_p = jnp.pad(table, ((0, _VPAD - (NUM_CLASSES + 1)), (0, 0)))
    return k(y2d, table_p)
